# baseline (speedup 1.0000x reference)
import functools

import jax
import jax.numpy as jnp
from jax import lax
from jax.experimental import pallas as pl
from jax.experimental.pallas import tpu as pltpu
from jax.experimental.pallas import tpu_sc as plsc

N = 100000
V = 61
D = 512
CH = 80
NCH = N // CH
NC = 2
NS = 16
NW = NC * NS
MAXCH = (NCH + NW - 1) // NW

_mesh = plsc.VectorSubcoreMesh(core_axis_name="c", subcore_axis_name="s")


@functools.partial(
    pl.kernel,
    out_type=jax.ShapeDtypeStruct((N * D,), jnp.float32),
    mesh=_mesh,
    scratch_types=[
        pltpu.VMEM((MAXCH, CH), jnp.int32),
        pltpu.VMEM_SHARED((V * D,), jnp.float32),
        pltpu.VMEM((2, CH * D), jnp.float32),
        pltpu.SemaphoreType.DMA((2,)),
        pltpu.SemaphoreType.DMA((2,)),
    ],
)
def _emb_lookup(idx_hbm, table_hbm, out_hbm, idx_v, table_sh, rows_v, ssem, gsem):
    wid = lax.axis_index("s") * NC + lax.axis_index("c")
    nchunks = (NCH - wid + NW - 1) // NW

    @pl.when(lax.axis_index("s") == 0)
    def _():
        pltpu.sync_copy(table_hbm, table_sh)

    pltpu.sync_copy(idx_hbm.at[:, wid], idx_v)
    plsc.subcore_barrier()

    def base_of(i):
        return (wid + i * NW) * CH

    def fill_rows(ci, b):
        def group_body(g16, carry):
            r0 = g16 * 16
            idx16 = idx_v[ci, pl.ds(r0, 16)]
            for j in range(16):
                row = idx16[j]
                pltpu.sync_copy(
                    table_sh.at[pl.ds(row * D, D)],
                    rows_v.at[b, pl.ds((r0 + j) * D, D)],
                )
            return carry

        lax.fori_loop(0, CH // 16, group_body, 0)

    def start_store(i, b):
        pltpu.make_async_copy(
            rows_v.at[b], out_hbm.at[pl.ds(base_of(i) * D, CH * D)], ssem.at[b]
        ).start()

    def wait_store(b):
        pltpu.make_async_copy(
            rows_v.at[b], out_hbm.at[pl.ds(0, CH * D)], ssem.at[b]
        ).wait()

    def body(g, carry):
        for b in (0, 1):
            i = 2 * g + b

            @pl.when(g > 0)
            def _():
                wait_store(b)

            fill_rows(i, b)
            start_store(i, b)
        return carry

    lax.fori_loop(0, nchunks // 2, body, 0)

    @pl.when(nchunks % 2 == 1)
    def _():
        wait_store(0)
        fill_rows(nchunks - 1, 0)
        start_store(nchunks - 1, 0)

    wait_store(0)
    wait_store(1)


def kernel(x, W0):
    idx = x.reshape(N).astype(jnp.int32)
    idx_pad = jnp.zeros((MAXCH * NW * CH,), jnp.int32).at[:N].set(idx)
    out = _emb_lookup(idx_pad.reshape(MAXCH, NW, CH), W0.reshape(V * D))
    return out.reshape(N, D)


# speedup vs baseline: 1.7787x; 1.7787x over previous
import functools

import jax
import jax.numpy as jnp
from jax import lax
from jax.experimental import pallas as pl
from jax.experimental.pallas import tpu as pltpu
from jax.experimental.pallas import tpu_sc as plsc

N = 100000
V = 61
D = 512
CH = 80
NCH = N // CH
NC = 2
NS = 16
NW = NC * NS
MAXCH = (NCH + NW - 1) // NW

_mesh = plsc.VectorSubcoreMesh(core_axis_name="c", subcore_axis_name="s")


@functools.partial(
    pl.kernel,
    out_type=jax.ShapeDtypeStruct((N * D,), jnp.float32),
    mesh=_mesh,
    scratch_types=[
        pltpu.VMEM((MAXCH, CH), jnp.int32),
        pltpu.VMEM_SHARED((V * D,), jnp.float32),
        pltpu.VMEM((2, CH * D), jnp.float32),
        pltpu.SemaphoreType.DMA((2,)),
        pltpu.SemaphoreType.DMA((2,)),
    ],
)
def _emb_lookup(idx_hbm, table_hbm, out_hbm, idx_v, table_sh, rows_v, ssem, gsem):
    wid = lax.axis_index("s") * NC + lax.axis_index("c")
    nchunks = (NCH - wid + NW - 1) // NW

    @pl.when(lax.axis_index("s") == 0)
    def _():
        pltpu.sync_copy(table_hbm, table_sh)

    pltpu.sync_copy(idx_hbm.at[:, wid], idx_v)
    plsc.subcore_barrier()

    def base_of(i):
        return (wid + i * NW) * CH

    def fill_rows(ci, b):
        def group_body(g16, carry):
            r0 = g16 * 16
            idx16 = idx_v[ci, pl.ds(r0, 16)]
            for j in range(16):
                row = idx16[j]
                pltpu.make_async_copy(
                    table_sh.at[pl.ds(row * D, D)],
                    rows_v.at[b, pl.ds((r0 + j) * D, D)],
                    gsem.at[b],
                ).start()
            for j in range(16):
                pltpu.make_async_copy(
                    table_sh.at[pl.ds(0, D)],
                    rows_v.at[b, pl.ds((r0 + j) * D, D)],
                    gsem.at[b],
                ).wait()
            return carry

        lax.fori_loop(0, CH // 16, group_body, 0)

    def start_store(i, b):
        pltpu.make_async_copy(
            rows_v.at[b], out_hbm.at[pl.ds(base_of(i) * D, CH * D)], ssem.at[b]
        ).start()

    def wait_store(b):
        pltpu.make_async_copy(
            rows_v.at[b], out_hbm.at[pl.ds(0, CH * D)], ssem.at[b]
        ).wait()

    def body(g, carry):
        for b in (0, 1):
            i = 2 * g + b

            @pl.when(g > 0)
            def _():
                wait_store(b)

            fill_rows(i, b)
            start_store(i, b)
        return carry

    lax.fori_loop(0, nchunks // 2, body, 0)

    @pl.when(nchunks % 2 == 1)
    def _():
        wait_store(0)
        fill_rows(nchunks - 1, 0)
        start_store(nchunks - 1, 0)

    wait_store(0)
    wait_store(1)


def kernel(x, W0):
    idx = x.reshape(N).astype(jnp.int32)
    idx_pad = jnp.zeros((MAXCH * NW * CH,), jnp.int32).at[:N].set(idx)
    out = _emb_lookup(idx_pad.reshape(MAXCH, NW, CH), W0.reshape(V * D))
    return out.reshape(N, D)
